# baseline (speedup 1.0000x reference)
import functools

import jax
import jax.numpy as jnp
from jax import lax
from jax.experimental import pallas as pl
from jax.experimental.pallas import tpu as pltpu
from jax.experimental.pallas import tpu_sc as plsc

_N = 10000
_NP = 10240
_D = 128
_SEQ = 50
_NTAG = 10
_TZ = 5
_F = 64
_NE = 320000
_NEP = 327680
_NC, _NS = 2, 16
_NW = _NC * _NS

_MESH = plsc.VectorSubcoreMesh(core_axis_name="c", subcore_axis_name="s")


def _sc_gather(table, idx, chunk, nbuf, k0=None):
    b = idx.shape[0]
    d = table.shape[1]
    b_per_w = b // _NW
    blk = chunk * nbuf
    pair_blocks = 2 * (b_per_w // blk)
    if k0 is None:
        k0 = pair_blocks // 2

    @functools.partial(
        pl.kernel, mesh=_MESH,
        out_type=jax.ShapeDtypeStruct((b, d), table.dtype),
        scratch_types=[
            pltpu.VMEM((blk,), jnp.int32),
            pltpu.VMEM((nbuf, chunk, d), table.dtype),
            pltpu.SemaphoreType.DMA,
            pltpu.SemaphoreType.DMA,
        ],
    )
    def k(table_hbm, idx_hbm, out_hbm, idx_v, rows_v, semg, semo):
        c = lax.axis_index("c")
        s = lax.axis_index("s")
        nblk = jnp.where(c == 0, k0, pair_blocks - k0)
        base = pl.multiple_of(
            s * 2 * b_per_w + jnp.where(c == 0, 0, k0 * blk), chunk)

        def body(i, carry):
            off = base + i * blk
            pltpu.sync_copy(idx_hbm.at[pl.ds(off, blk)], idx_v)
            gd = [pltpu.async_copy(
                table_hbm.at[idx_v.at[pl.ds(j * chunk, chunk)]],
                rows_v.at[j], semg) for j in range(nbuf)]
            od = []
            for j in range(nbuf):
                gd[j].wait()
                od.append(pltpu.async_copy(
                    rows_v.at[j],
                    out_hbm.at[pl.ds(off + j * chunk, chunk)], semo))
            for o in od:
                o.wait()
            return carry

        lax.fori_loop(0, nblk, body, 0)

    return k(table, idx)


def _sc_gather_sum5(table, idx, k0=None):
    b = idx.shape[0]
    d = table.shape[1]
    chunk = 80
    nbuf = 5
    blk = chunk * nbuf
    grp = chunk // _TZ
    b_per_w = b // _NW
    g_per_w = b_per_w // _TZ
    pair_blocks = 2 * (b_per_w // blk)
    if k0 is None:
        k0 = pair_blocks // 2

    @functools.partial(
        pl.kernel, mesh=_MESH,
        out_type=jax.ShapeDtypeStruct((b // _TZ, d), table.dtype),
        scratch_types=[
            pltpu.VMEM((blk,), jnp.int32),
            pltpu.VMEM((nbuf, chunk, d), table.dtype),
            pltpu.VMEM((blk // _TZ, d), table.dtype),
            pltpu.SemaphoreType.DMA,
        ],
    )
    def k(table_hbm, idx_hbm, out_hbm, idx_v, rows_v, sum_v, sem):
        c = lax.axis_index("c")
        s = lax.axis_index("s")
        nblk = jnp.where(c == 0, k0, pair_blocks - k0)
        base = pl.multiple_of(
            s * 2 * b_per_w + jnp.where(c == 0, 0, k0 * blk), 80)
        gbase = pl.multiple_of(base // _TZ, 16)

        def body(i, carry):
            pltpu.sync_copy(idx_hbm.at[pl.ds(base + i * blk, blk)], idx_v)
            gd = [pltpu.async_copy(
                table_hbm.at[idx_v.at[pl.ds(j * chunk, chunk)]],
                rows_v.at[j], sem) for j in range(nbuf)]
            for j in range(nbuf):
                gd[j].wait()

                def gsum(g, carry2):
                    for cb in range(d // 16):
                        sl = pl.ds(cb * 16, 16)
                        acc = rows_v[j, 5 * g, sl]
                        for z in range(1, _TZ):
                            acc = acc + rows_v[j, 5 * g + z, sl]
                        sum_v[j * grp + g, sl] = acc
                    return carry2

                lax.fori_loop(0, grp, gsum, 0)
            pltpu.sync_copy(sum_v,
                            out_hbm.at[pl.ds(gbase + i * (blk // _TZ),
                                             blk // _TZ)])
            return carry

        lax.fori_loop(0, nblk, body, 0)

    return k(table, idx)


_HN = _NP // 2
_ACC = 5248
_EC = 128
_ENB = 4
_ZC = _ACC // _NS // 4


def _zero_acc(zeros_hbm, z_v, acc_sh, s):
    pltpu.sync_copy(zeros_hbm, z_v)
    for q in range(4):
        pltpu.sync_copy(z_v, acc_sh.at[pl.ds((s * 4 + q) * _ZC, _ZC)])


def _remap_local(dst_v, j, lo):
    for kk in range(_EC // 16):
        sl = pl.ds(kk * 16, 16)
        loc = dst_v[j, sl] - lo
        ok = (loc >= 0) & (loc < _HN)
        dst_v[j, sl] = jnp.where(ok, loc, _HN)


def _sc_degree(dst2, ones_rows, zeros_rows):
    rows_pt = dst2.shape[0] // _NS
    nblk = rows_pt // _ENB
    orows_t = _HN // _NS

    @functools.partial(
        pl.kernel, mesh=_MESH,
        out_type=jax.ShapeDtypeStruct((_NC, _HN, _D), jnp.float32),
        scratch_types=[
            pltpu.VMEM((_ENB, _EC), jnp.int32),
            pltpu.VMEM((_EC, _D), jnp.float32),
            pltpu.VMEM((_ZC, _D), jnp.float32),
            pltpu.VMEM_SHARED((_ACC, _D), jnp.float32),
            pltpu.SemaphoreType.DMA,
        ],
    )
    def k(dst_hbm, ones_hbm, zeros_hbm, out_hbm, dst_v, ones_v, z_v, acc_sh,
          sem):
        c = lax.axis_index("c")
        s = lax.axis_index("s")
        base = s * rows_pt
        lo = c * _HN
        pltpu.sync_copy(ones_hbm, ones_v)
        _zero_acc(zeros_hbm, z_v, acc_sh, s)
        plsc.subcore_barrier()

        def body(i, carry):
            pltpu.sync_copy(dst_hbm.at[pl.ds(base + i * _ENB, _ENB)], dst_v)
            sd = []
            for j in range(_ENB):
                _remap_local(dst_v, j, lo)
                sd.append(pltpu.async_copy(
                    ones_v, acc_sh.at[dst_v.at[j]], sem, add=True))
            for o in sd:
                o.wait()
            return carry

        lax.fori_loop(0, nblk, body, 0)
        plsc.subcore_barrier()
        pltpu.sync_copy(acc_sh.at[pl.ds(s * orows_t, orows_t)],
                        out_hbm.at[c, pl.ds(s * orows_t, orows_t)])

    return k(dst2, ones_rows, zeros_rows)


def _sc_edge_segsum(g, src2, dst2, zeros_rows):
    d = g.shape[1]
    rows_pt = src2.shape[0] // _NS
    nblk = rows_pt // _ENB
    orows_t = _HN // _NS

    @functools.partial(
        pl.kernel, mesh=_MESH,
        out_type=jax.ShapeDtypeStruct((_NC, _HN, d), jnp.float32),
        scratch_types=[
            pltpu.VMEM((_ENB, _EC), jnp.int32),
            pltpu.VMEM((_ENB, _EC), jnp.int32),
            pltpu.VMEM((_ENB, _EC, d), jnp.float32),
            pltpu.VMEM((_ZC, d), jnp.float32),
            pltpu.VMEM_SHARED((_ACC, d), jnp.float32),
            pltpu.SemaphoreType.DMA,
            pltpu.SemaphoreType.DMA,
        ],
    )
    def k(g_hbm, src_hbm, dst_hbm, zeros_hbm, out_hbm,
          src_v, dst_v, rows_v, z_v, acc_sh, semg, semo):
        c = lax.axis_index("c")
        s = lax.axis_index("s")
        base = s * rows_pt
        lo = c * _HN
        _zero_acc(zeros_hbm, z_v, acc_sh, s)
        plsc.subcore_barrier()

        def body(i, carry):
            off = base + i * _ENB
            pltpu.sync_copy(src_hbm.at[pl.ds(off, _ENB)], src_v)
            pltpu.sync_copy(dst_hbm.at[pl.ds(off, _ENB)], dst_v)
            gd = [pltpu.async_copy(g_hbm.at[src_v.at[j]], rows_v.at[j], semg)
                  for j in range(_ENB)]
            sd = []
            for j in range(_ENB):
                _remap_local(dst_v, j, lo)
                gd[j].wait()
                sd.append(pltpu.async_copy(
                    rows_v.at[j], acc_sh.at[dst_v.at[j]], semo, add=True))
            for o in sd:
                o.wait()
            return carry

        lax.fori_loop(0, nblk, body, 0)
        plsc.subcore_barrier()
        pltpu.sync_copy(acc_sh.at[pl.ds(s * orows_t, orows_t)],
                        out_hbm.at[c, pl.ds(s * orows_t, orows_t)])

    return k(g, src2, dst2, zeros_rows)


_BLK = 128


def _tc_conv_kernel(e_ref, wcat_ref, b1_ref, b2_ref, x_ref):
    eb = e_ref[...].astype(jnp.bfloat16)
    a = jnp.dot(eb, wcat_ref[...], preferred_element_type=jnp.float32)
    a3 = a.reshape(_BLK, _SEQ, 6 * _F)
    p1 = a3[:, :_SEQ - 1, 0:_F] + a3[:, 1:, _F:2 * _F]
    m1 = jnp.max(p1, axis=1)
    p2 = (a3[:, :_SEQ - 3, 2 * _F:3 * _F] + a3[:, 1:_SEQ - 2, 3 * _F:4 * _F]
          + a3[:, 2:_SEQ - 1, 4 * _F:5 * _F] + a3[:, 3:, 5 * _F:6 * _F])
    m2 = jnp.max(p2, axis=1)
    x1 = jnp.maximum(m1 + b1_ref[...], 0.0)
    x2 = jnp.maximum(m2 + b2_ref[...], 0.0)
    x_ref[...] = jnp.concatenate([x1, x2], axis=1)


def _tc_conv(e2, wcat, b1, b2):
    n_blk = _NP // _BLK
    return pl.pallas_call(
        _tc_conv_kernel,
        grid=(n_blk,),
        in_specs=[
            pl.BlockSpec((_BLK * _SEQ, _D), lambda i: (i, 0)),
            pl.BlockSpec((_D, 6 * _F), lambda i: (0, 0)),
            pl.BlockSpec((1, _F), lambda i: (0, 0)),
            pl.BlockSpec((1, _F), lambda i: (0, 0)),
        ],
        out_specs=pl.BlockSpec((_BLK, 2 * _F), lambda i: (i, 0)),
        out_shape=jax.ShapeDtypeStruct((_NP, 2 * _F), jnp.float32),
    )(e2, wcat, b1, b2)


def _tc_tagfc_kernel(ts_ref, fc1t_ref, fc2t_ref, t_ref):
    ts = ts_ref[...]
    r1 = jnp.maximum(jnp.dot(ts, fc1t_ref[...],
                             preferred_element_type=jnp.float32), 0.0)
    r2 = jnp.maximum(jnp.dot(ts, fc2t_ref[...],
                             preferred_element_type=jnp.float32), 0.0)
    r1 = r1.reshape(_BLK, _NTAG, _F)
    r2 = r2.reshape(_BLK, _NTAG, _F)
    t1 = jnp.sum(r1, axis=1)
    t2 = jnp.sum(r2, axis=1)
    t_ref[...] = jnp.concatenate([t1, t2], axis=1)


def _tc_tagfc(ts2, fc1t, fc2t):
    n_blk = _NP // _BLK
    return pl.pallas_call(
        _tc_tagfc_kernel,
        grid=(n_blk,),
        in_specs=[
            pl.BlockSpec((_BLK * _NTAG, _D), lambda i: (i, 0)),
            pl.BlockSpec((_D, _F), lambda i: (0, 0)),
            pl.BlockSpec((_D, _F), lambda i: (0, 0)),
        ],
        out_specs=pl.BlockSpec((_BLK, 2 * _F), lambda i: (i, 0)),
        out_shape=jax.ShapeDtypeStruct((_NP, 2 * _F), jnp.float32),
    )(ts2, fc1t, fc2t)


def _bn_cols(x, mask, gamma, beta):
    cnt = float(_N)
    xm = jnp.sum(x * mask, axis=0, keepdims=True) / cnt
    xv = jnp.sum(x * x * mask, axis=0, keepdims=True) / cnt - xm * xm
    return (x - xm) * lax.rsqrt(xv + 1e-5) * gamma + beta


def _dinv_col(degp_ref):
    deg = jnp.concatenate(
        [degp_ref[0, :, 0:1], degp_ref[1, :, 0:1]], axis=0) + 1.0
    return lax.rsqrt(deg)


def _tc_bn_combine_kernel(x_ref, t_ref, degp_ref, g1_ref, be1_ref,
                          g2_ref, be2_ref, w1_ref, hw_ref, gv_ref):
    mask = (lax.broadcasted_iota(jnp.int32, (_NP, 1), 0) < _N
            ).astype(jnp.float32)
    xb = _bn_cols(x_ref[...], mask, g1_ref[...], be1_ref[...])
    tb = _bn_cols(t_ref[...], mask, g2_ref[...], be2_ref[...])
    h = xb + tb
    dinv = _dinv_col(degp_ref)
    hw = jnp.dot(h, w1_ref[...], preferred_element_type=jnp.float32)
    hw_ref[...] = hw
    gv_ref[...] = hw * dinv


def _tc_bn_combine(x, t, degp, bn1g, bn1b, bn2g, bn2b, gcn1_w):
    sds = jax.ShapeDtypeStruct((_NP, _D), jnp.float32)
    return pl.pallas_call(
        _tc_bn_combine_kernel,
        out_shape=[sds, sds],
    )(x, t, degp, bn1g, bn1b, bn2g, bn2b, gcn1_w)


def _tc_gcn_mid_kernel(s_ref, hw_ref, degp_ref, b_ref, w2_ref,
                       hw2_ref, gv_ref):
    dinv = _dinv_col(degp_ref)
    agg = jnp.concatenate([s_ref[0], s_ref[1]], axis=0)
    h1 = dinv * agg + dinv * dinv * hw_ref[...] + b_ref[...]
    h1 = jnp.maximum(h1, 0.0)
    hw2 = jnp.dot(h1, w2_ref[...], preferred_element_type=jnp.float32)
    hw2_ref[...] = hw2
    gv_ref[...] = hw2 * dinv


def _tc_gcn_mid(s, hw, degp, b, w2):
    sds = jax.ShapeDtypeStruct((_NP, _D), jnp.float32)
    return pl.pallas_call(
        _tc_gcn_mid_kernel,
        out_shape=[sds, sds],
    )(s, hw, degp, b, w2)


def _tc_gcn_out_kernel(s_ref, hw_ref, degp_ref, b_ref, emb_ref):
    dinv = _dinv_col(degp_ref)
    agg = jnp.concatenate([s_ref[0], s_ref[1]], axis=0)
    h2 = dinv * agg + dinv * dinv * hw_ref[...] + b_ref[...]
    nrm = jnp.sqrt(jnp.sum(h2 * h2, axis=1, keepdims=True))
    emb_ref[...] = h2 / jnp.maximum(nrm, 1e-12)


def _tc_gcn_out(s, hw, degp, b):
    return pl.pallas_call(
        _tc_gcn_out_kernel,
        out_shape=jax.ShapeDtypeStruct((_NP, _D), jnp.float32),
    )(s, hw, degp, b)


def kernel(repo_index, repo_features, repo_tags, bridge_ids, repo_edge_index,
           embed_table, conv1_w, conv1_b, conv2_w, conv2_b, bn1_gamma,
           bn1_beta, fc1_w, fc2_w, bn2_gamma, bn2_beta, gcn1_w, gcn1_b,
           gcn2_w, gcn2_b):
    bn, kk = repo_index.shape

    feat_idx = jnp.concatenate(
        [repo_features,
         jnp.zeros((_NP - _N, _SEQ), jnp.int32)], axis=0).reshape(-1)
    tag_idx = jnp.concatenate(
        [repo_tags,
         jnp.zeros((_NP - _N, _NTAG, _TZ), jnp.int32)], axis=0).reshape(-1)
    src2 = jnp.concatenate(
        [repo_edge_index[0],
         jnp.zeros((_NEP - _NE,), jnp.int32)]).reshape(_NEP // _EC, _EC)
    dst2 = jnp.concatenate(
        [repo_edge_index[1],
         jnp.full((_NEP - _NE,), _N, jnp.int32)]).reshape(_NEP // _EC, _EC)

    wcat = jnp.concatenate(
        [conv1_w[:, 0, 0, :].T, conv1_w[:, 0, 1, :].T,
         conv2_w[:, 0, 0, :].T, conv2_w[:, 0, 1, :].T,
         conv2_w[:, 0, 2, :].T, conv2_w[:, 0, 3, :].T], axis=1)
    b1 = conv1_b.reshape(1, _F)
    b2 = conv2_b.reshape(1, _F)
    fc1t = fc1_w.T
    fc2t = fc2_w.T
    bn1g = bn1_gamma.reshape(1, 2 * _F)
    bn1b = bn1_beta.reshape(1, 2 * _F)
    bn2g = bn2_gamma.reshape(1, 2 * _F)
    bn2b = bn2_beta.reshape(1, 2 * _F)
    g1b = gcn1_b.reshape(1, _D)
    g2b = gcn2_b.reshape(1, _D)

    zeros_acc = jnp.zeros((_ZC, _D), jnp.float32)
    ones_d = jnp.ones((_EC, _D), jnp.float32)

    e2 = _sc_gather(embed_table, feat_idx, 128, 5, k0=13)
    x = _tc_conv(e2, wcat.astype(jnp.bfloat16), b1, b2)
    dep = (e2[0, 0] * 0.0).astype(jnp.int32)
    ts2 = _sc_gather_sum5(embed_table, tag_idx + dep, k0=21)
    t = _tc_tagfc(ts2, fc1t, fc2t)
    degp = _sc_degree(dst2, ones_d, zeros_acc)

    hw1, g1v = _tc_bn_combine(x, t, degp, bn1g, bn1b, bn2g, bn2b, gcn1_w)

    s1 = _sc_edge_segsum(g1v, src2, dst2, zeros_acc)
    hw2, g2v = _tc_gcn_mid(s1, hw1, degp, g1b, gcn2_w)

    s2 = _sc_edge_segsum(g2v, src2, dst2, zeros_acc)
    all_emb = _tc_gcn_out(s2, hw2, degp, g2b)

    fin_idx = jnp.concatenate([bridge_ids, repo_index.reshape(-1)])
    fin = _sc_gather(all_emb, fin_idx, 96, 6)
    bridges = fin[:bridge_ids.shape[0]]
    sel = fin[bridge_ids.shape[0]:].reshape(bn, kk, _D)
    return bridges, sel


# speedup vs baseline: 1.1291x; 1.1291x over previous
import functools

import jax
import jax.numpy as jnp
from jax import lax
from jax.experimental import pallas as pl
from jax.experimental.pallas import tpu as pltpu
from jax.experimental.pallas import tpu_sc as plsc

_N = 10000
_NP = 10240
_D = 128
_SEQ = 50
_NTAG = 10
_TZ = 5
_F = 64
_NE = 320000
_NEP = 327680
_NC, _NS = 2, 16
_NW = _NC * _NS

_MESH = plsc.VectorSubcoreMesh(core_axis_name="c", subcore_axis_name="s")


def _sc_gather(table, idx, chunk, nbuf, k0=None):
    b = idx.shape[0]
    d = table.shape[1]
    b_per_w = b // _NW
    blk = chunk * nbuf
    pair_blocks = 2 * (b_per_w // blk)
    if k0 is None:
        k0 = pair_blocks // 2

    @functools.partial(
        pl.kernel, mesh=_MESH,
        out_type=jax.ShapeDtypeStruct((b, d), table.dtype),
        scratch_types=[
            pltpu.VMEM((blk,), jnp.int32),
            pltpu.VMEM((nbuf, chunk, d), table.dtype),
            pltpu.SemaphoreType.DMA,
            pltpu.SemaphoreType.DMA,
        ],
    )
    def k(table_hbm, idx_hbm, out_hbm, idx_v, rows_v, semg, semo):
        c = lax.axis_index("c")
        s = lax.axis_index("s")
        nblk = jnp.where(c == 0, k0, pair_blocks - k0)
        base = pl.multiple_of(
            s * 2 * b_per_w + jnp.where(c == 0, 0, k0 * blk), chunk)

        def body(i, carry):
            off = base + i * blk
            pltpu.sync_copy(idx_hbm.at[pl.ds(off, blk)], idx_v)
            gd = [pltpu.async_copy(
                table_hbm.at[idx_v.at[pl.ds(j * chunk, chunk)]],
                rows_v.at[j], semg) for j in range(nbuf)]
            od = []
            for j in range(nbuf):
                gd[j].wait()
                od.append(pltpu.async_copy(
                    rows_v.at[j],
                    out_hbm.at[pl.ds(off + j * chunk, chunk)], semo))
            for o in od:
                o.wait()
            return carry

        lax.fori_loop(0, nblk, body, 0)

    return k(table, idx)


def _sc_gather_sum5(table, idx, k0=None):
    b = idx.shape[0]
    d = table.shape[1]
    chunk = 80
    nbuf = 5
    blk = chunk * nbuf
    grp = chunk // _TZ
    b_per_w = b // _NW
    g_per_w = b_per_w // _TZ
    pair_blocks = 2 * (b_per_w // blk)
    if k0 is None:
        k0 = pair_blocks // 2

    @functools.partial(
        pl.kernel, mesh=_MESH,
        out_type=jax.ShapeDtypeStruct((b // _TZ, d), table.dtype),
        scratch_types=[
            pltpu.VMEM((blk,), jnp.int32),
            pltpu.VMEM((nbuf, chunk, d), table.dtype),
            pltpu.VMEM((blk // _TZ, d), table.dtype),
            pltpu.SemaphoreType.DMA,
        ],
    )
    def k(table_hbm, idx_hbm, out_hbm, idx_v, rows_v, sum_v, sem):
        c = lax.axis_index("c")
        s = lax.axis_index("s")
        nblk = jnp.where(c == 0, k0, pair_blocks - k0)
        base = pl.multiple_of(
            s * 2 * b_per_w + jnp.where(c == 0, 0, k0 * blk), 80)
        gbase = pl.multiple_of(base // _TZ, 16)

        def body(i, carry):
            pltpu.sync_copy(idx_hbm.at[pl.ds(base + i * blk, blk)], idx_v)
            gd = [pltpu.async_copy(
                table_hbm.at[idx_v.at[pl.ds(j * chunk, chunk)]],
                rows_v.at[j], sem) for j in range(nbuf)]
            for j in range(nbuf):
                gd[j].wait()

                def gsum(g, carry2):
                    for cb in range(d // 16):
                        sl = pl.ds(cb * 16, 16)
                        acc = rows_v[j, 5 * g, sl]
                        for z in range(1, _TZ):
                            acc = acc + rows_v[j, 5 * g + z, sl]
                        sum_v[j * grp + g, sl] = acc
                    return carry2

                lax.fori_loop(0, grp, gsum, 0)
            pltpu.sync_copy(sum_v,
                            out_hbm.at[pl.ds(gbase + i * (blk // _TZ),
                                             blk // _TZ)])
            return carry

        lax.fori_loop(0, nblk, body, 0)

    return k(table, idx)


_HN = _NP // 2
_ACC = 5248
_EC = 128
_ENB = 4
_ZC = _ACC // _NS // 4


def _zero_acc(zeros_hbm, z_v, acc_sh, s):
    pltpu.sync_copy(zeros_hbm, z_v)
    for q in range(4):
        pltpu.sync_copy(z_v, acc_sh.at[pl.ds((s * 4 + q) * _ZC, _ZC)])


def _remap_local(dst_v, j, lo):
    for kk in range(_EC // 16):
        sl = pl.ds(kk * 16, 16)
        loc = dst_v[j, sl] - lo
        ok = (loc >= 0) & (loc < _HN)
        dst_v[j, sl] = jnp.where(ok, loc, _HN)


def _sc_degree(dst2, ones_rows, zeros_rows):
    rows_pt = dst2.shape[0] // _NS
    nblk = rows_pt // _ENB
    orows_t = _HN // _NS

    @functools.partial(
        pl.kernel, mesh=_MESH,
        out_type=jax.ShapeDtypeStruct((_NC, _HN, _D), jnp.float32),
        scratch_types=[
            pltpu.VMEM((_ENB, _EC), jnp.int32),
            pltpu.VMEM((_EC, _D), jnp.float32),
            pltpu.VMEM((_ZC, _D), jnp.float32),
            pltpu.VMEM_SHARED((_ACC, _D), jnp.float32),
            pltpu.SemaphoreType.DMA,
        ],
    )
    def k(dst_hbm, ones_hbm, zeros_hbm, out_hbm, dst_v, ones_v, z_v, acc_sh,
          sem):
        c = lax.axis_index("c")
        s = lax.axis_index("s")
        base = s * rows_pt
        lo = c * _HN
        pltpu.sync_copy(ones_hbm, ones_v)
        _zero_acc(zeros_hbm, z_v, acc_sh, s)
        plsc.subcore_barrier()

        def body(i, carry):
            pltpu.sync_copy(dst_hbm.at[pl.ds(base + i * _ENB, _ENB)], dst_v)
            sd = []
            for j in range(_ENB):
                _remap_local(dst_v, j, lo)
                sd.append(pltpu.async_copy(
                    ones_v, acc_sh.at[dst_v.at[j]], sem, add=True))
            for o in sd:
                o.wait()
            return carry

        lax.fori_loop(0, nblk, body, 0)
        plsc.subcore_barrier()
        pltpu.sync_copy(acc_sh.at[pl.ds(s * orows_t, orows_t)],
                        out_hbm.at[c, pl.ds(s * orows_t, orows_t)])

    return k(dst2, ones_rows, zeros_rows)


def _sc_edge_segsum(g, src2, dst2, zeros_rows):
    d = g.shape[1]
    rows_pt = src2.shape[0] // _NS
    nblk = rows_pt // _ENB
    orows_t = _HN // _NS

    @functools.partial(
        pl.kernel, mesh=_MESH,
        out_type=jax.ShapeDtypeStruct((_NC, _HN, d), jnp.float32),
        scratch_types=[
            pltpu.VMEM((_ENB, _EC), jnp.int32),
            pltpu.VMEM((_ENB, _EC), jnp.int32),
            pltpu.VMEM((_ENB, _EC, d), jnp.float32),
            pltpu.VMEM((_ZC, d), jnp.float32),
            pltpu.VMEM_SHARED((_ACC, d), jnp.float32),
            pltpu.SemaphoreType.DMA,
            pltpu.SemaphoreType.DMA,
        ],
    )
    def k(g_hbm, src_hbm, dst_hbm, zeros_hbm, out_hbm,
          src_v, dst_v, rows_v, z_v, acc_sh, semg, semo):
        c = lax.axis_index("c")
        s = lax.axis_index("s")
        base = s * rows_pt
        lo = c * _HN
        _zero_acc(zeros_hbm, z_v, acc_sh, s)
        plsc.subcore_barrier()

        def body(i, carry):
            off = base + i * _ENB
            pltpu.sync_copy(src_hbm.at[pl.ds(off, _ENB)], src_v)
            pltpu.sync_copy(dst_hbm.at[pl.ds(off, _ENB)], dst_v)
            gd = [pltpu.async_copy(g_hbm.at[src_v.at[j]], rows_v.at[j], semg)
                  for j in range(_ENB)]
            sd = []
            for j in range(_ENB):
                _remap_local(dst_v, j, lo)
                gd[j].wait()
                sd.append(pltpu.async_copy(
                    rows_v.at[j], acc_sh.at[dst_v.at[j]], semo, add=True))
            for o in sd:
                o.wait()
            return carry

        lax.fori_loop(0, nblk, body, 0)
        plsc.subcore_barrier()
        pltpu.sync_copy(acc_sh.at[pl.ds(s * orows_t, orows_t)],
                        out_hbm.at[c, pl.ds(s * orows_t, orows_t)])

    return k(g, src2, dst2, zeros_rows)


_BLK = 128


def _tc_conv_kernel(e_ref, wcat_ref, b1_ref, b2_ref, x_ref):
    eb = e_ref[...].astype(jnp.bfloat16)
    a = jnp.dot(eb, wcat_ref[...], preferred_element_type=jnp.float32)
    a3 = a.reshape(_BLK, _SEQ, 6 * _F)
    p1 = a3[:, :_SEQ - 1, 0:_F] + a3[:, 1:, _F:2 * _F]
    m1 = jnp.max(p1, axis=1)
    p2 = (a3[:, :_SEQ - 3, 2 * _F:3 * _F] + a3[:, 1:_SEQ - 2, 3 * _F:4 * _F]
          + a3[:, 2:_SEQ - 1, 4 * _F:5 * _F] + a3[:, 3:, 5 * _F:6 * _F])
    m2 = jnp.max(p2, axis=1)
    x1 = jnp.maximum(m1 + b1_ref[...], 0.0)
    x2 = jnp.maximum(m2 + b2_ref[...], 0.0)
    x_ref[...] = jnp.concatenate([x1, x2], axis=1)


def _tc_conv(e2, wcat, b1, b2):
    n_blk = _NP // _BLK
    return pl.pallas_call(
        _tc_conv_kernel,
        grid=(n_blk,),
        in_specs=[
            pl.BlockSpec((_BLK * _SEQ, _D), lambda i: (i, 0)),
            pl.BlockSpec((_D, 6 * _F), lambda i: (0, 0)),
            pl.BlockSpec((1, _F), lambda i: (0, 0)),
            pl.BlockSpec((1, _F), lambda i: (0, 0)),
        ],
        out_specs=pl.BlockSpec((_BLK, 2 * _F), lambda i: (i, 0)),
        out_shape=jax.ShapeDtypeStruct((_NP, 2 * _F), jnp.float32),
    )(e2, wcat, b1, b2)


def _tc_tagfc_kernel(ts_ref, fc1t_ref, fc2t_ref, t_ref):
    ts = ts_ref[...]
    r1 = jnp.maximum(jnp.dot(ts, fc1t_ref[...],
                             preferred_element_type=jnp.float32), 0.0)
    r2 = jnp.maximum(jnp.dot(ts, fc2t_ref[...],
                             preferred_element_type=jnp.float32), 0.0)
    r1 = r1.reshape(_BLK, _NTAG, _F)
    r2 = r2.reshape(_BLK, _NTAG, _F)
    t1 = jnp.sum(r1, axis=1)
    t2 = jnp.sum(r2, axis=1)
    t_ref[...] = jnp.concatenate([t1, t2], axis=1)


def _tc_tagfc(ts2, fc1t, fc2t):
    n_blk = _NP // _BLK
    return pl.pallas_call(
        _tc_tagfc_kernel,
        grid=(n_blk,),
        in_specs=[
            pl.BlockSpec((_BLK * _NTAG, _D), lambda i: (i, 0)),
            pl.BlockSpec((_D, _F), lambda i: (0, 0)),
            pl.BlockSpec((_D, _F), lambda i: (0, 0)),
        ],
        out_specs=pl.BlockSpec((_BLK, 2 * _F), lambda i: (i, 0)),
        out_shape=jax.ShapeDtypeStruct((_NP, 2 * _F), jnp.float32),
    )(ts2, fc1t, fc2t)


def _bn_cols(x, mask, gamma, beta):
    cnt = float(_N)
    xm = jnp.sum(x * mask, axis=0, keepdims=True) / cnt
    xv = jnp.sum(x * x * mask, axis=0, keepdims=True) / cnt - xm * xm
    return (x - xm) * lax.rsqrt(xv + 1e-5) * gamma + beta


def _dinv_col(degp_ref):
    deg = jnp.concatenate(
        [degp_ref[0, :, 0:1], degp_ref[1, :, 0:1]], axis=0) + 1.0
    return lax.rsqrt(deg)


def _tc_bn_combine_kernel(x_ref, t_ref, degp_ref, g1_ref, be1_ref,
                          g2_ref, be2_ref, w1_ref, hw_ref, gv_ref):
    mask = (lax.broadcasted_iota(jnp.int32, (_NP, 1), 0) < _N
            ).astype(jnp.float32)
    xb = _bn_cols(x_ref[...], mask, g1_ref[...], be1_ref[...])
    tb = _bn_cols(t_ref[...], mask, g2_ref[...], be2_ref[...])
    h = xb + tb
    dinv = _dinv_col(degp_ref)
    hw = jnp.dot(h, w1_ref[...], preferred_element_type=jnp.float32)
    hw_ref[...] = hw
    gv_ref[...] = hw * dinv


def _tc_bn_combine(x, t, degp, bn1g, bn1b, bn2g, bn2b, gcn1_w):
    sds = jax.ShapeDtypeStruct((_NP, _D), jnp.float32)
    return pl.pallas_call(
        _tc_bn_combine_kernel,
        out_shape=[sds, sds],
    )(x, t, degp, bn1g, bn1b, bn2g, bn2b, gcn1_w)


def _tc_gcn_mid_kernel(s_ref, hw_ref, degp_ref, b_ref, w2_ref,
                       hw2_ref, gv_ref):
    dinv = _dinv_col(degp_ref)
    agg = jnp.concatenate([s_ref[0], s_ref[1]], axis=0)
    h1 = dinv * agg + dinv * dinv * hw_ref[...] + b_ref[...]
    h1 = jnp.maximum(h1, 0.0)
    hw2 = jnp.dot(h1, w2_ref[...], preferred_element_type=jnp.float32)
    hw2_ref[...] = hw2
    gv_ref[...] = hw2 * dinv


def _tc_gcn_mid(s, hw, degp, b, w2):
    sds = jax.ShapeDtypeStruct((_NP, _D), jnp.float32)
    return pl.pallas_call(
        _tc_gcn_mid_kernel,
        out_shape=[sds, sds],
    )(s, hw, degp, b, w2)


def _tc_gcn_out_kernel(s_ref, hw_ref, degp_ref, b_ref, emb_ref):
    dinv = _dinv_col(degp_ref)
    agg = jnp.concatenate([s_ref[0], s_ref[1]], axis=0)
    h2 = dinv * agg + dinv * dinv * hw_ref[...] + b_ref[...]
    nrm = jnp.sqrt(jnp.sum(h2 * h2, axis=1, keepdims=True))
    emb_ref[...] = h2 / jnp.maximum(nrm, 1e-12)


def _tc_gcn_out(s, hw, degp, b):
    return pl.pallas_call(
        _tc_gcn_out_kernel,
        out_shape=jax.ShapeDtypeStruct((_NP, _D), jnp.float32),
    )(s, hw, degp, b)


def kernel(repo_index, repo_features, repo_tags, bridge_ids, repo_edge_index,
           embed_table, conv1_w, conv1_b, conv2_w, conv2_b, bn1_gamma,
           bn1_beta, fc1_w, fc2_w, bn2_gamma, bn2_beta, gcn1_w, gcn1_b,
           gcn2_w, gcn2_b):
    bn, kk = repo_index.shape

    feat_idx = jnp.concatenate(
        [repo_features,
         jnp.zeros((_NP - _N, _SEQ), jnp.int32)], axis=0).reshape(-1)
    tag_idx = jnp.concatenate(
        [repo_tags,
         jnp.zeros((_NP - _N, _NTAG, _TZ), jnp.int32)], axis=0).reshape(-1)
    src2 = jnp.concatenate(
        [repo_edge_index[0],
         jnp.zeros((_NEP - _NE,), jnp.int32)]).reshape(_NEP // _EC, _EC)
    dst2 = jnp.concatenate(
        [repo_edge_index[1],
         jnp.full((_NEP - _NE,), _N, jnp.int32)]).reshape(_NEP // _EC, _EC)

    wcat = jnp.concatenate(
        [conv1_w[:, 0, 0, :].T, conv1_w[:, 0, 1, :].T,
         conv2_w[:, 0, 0, :].T, conv2_w[:, 0, 1, :].T,
         conv2_w[:, 0, 2, :].T, conv2_w[:, 0, 3, :].T], axis=1)
    b1 = conv1_b.reshape(1, _F)
    b2 = conv2_b.reshape(1, _F)
    fc1t = fc1_w.T
    fc2t = fc2_w.T
    bn1g = bn1_gamma.reshape(1, 2 * _F)
    bn1b = bn1_beta.reshape(1, 2 * _F)
    bn2g = bn2_gamma.reshape(1, 2 * _F)
    bn2b = bn2_beta.reshape(1, 2 * _F)
    g1b = gcn1_b.reshape(1, _D)
    g2b = gcn2_b.reshape(1, _D)

    zeros_acc = jnp.zeros((_ZC, _D), jnp.float32)
    ones_d = jnp.ones((_EC, _D), jnp.float32)

    e2 = _sc_gather(embed_table, feat_idx, 128, 5, k0=37)
    x = _tc_conv(e2, wcat.astype(jnp.bfloat16), b1, b2)
    dep = (e2[0, 0] * 0.0).astype(jnp.int32)
    ts2 = _sc_gather_sum5(embed_table, tag_idx + dep, k0=59)
    t = _tc_tagfc(ts2, fc1t, fc2t)
    degp = _sc_degree(dst2, ones_d, zeros_acc)

    hw1, g1v = _tc_bn_combine(x, t, degp, bn1g, bn1b, bn2g, bn2b, gcn1_w)

    s1 = _sc_edge_segsum(g1v, src2, dst2, zeros_acc)
    hw2, g2v = _tc_gcn_mid(s1, hw1, degp, g1b, gcn2_w)

    s2 = _sc_edge_segsum(g2v, src2, dst2, zeros_acc)
    all_emb = _tc_gcn_out(s2, hw2, degp, g2b)

    fin_idx = jnp.concatenate([bridge_ids, repo_index.reshape(-1)])
    fin = _sc_gather(all_emb, fin_idx, 96, 6)
    bridges = fin[:bridge_ids.shape[0]]
    sel = fin[bridge_ids.shape[0]:].reshape(bn, kk, _D)
    return bridges, sel


# speedup vs baseline: 1.1431x; 1.0123x over previous
import functools

import jax
import jax.numpy as jnp
from jax import lax
from jax.experimental import pallas as pl
from jax.experimental.pallas import tpu as pltpu
from jax.experimental.pallas import tpu_sc as plsc

_N = 10000
_NP = 10240
_D = 128
_SEQ = 50
_NTAG = 10
_TZ = 5
_F = 64
_NE = 320000
_NEP = 327680
_NC, _NS = 2, 16
_NW = _NC * _NS

_MESH = plsc.VectorSubcoreMesh(core_axis_name="c", subcore_axis_name="s")


def _sc_gather(table, idx, chunk, nbuf, k0=None):
    b = idx.shape[0]
    d = table.shape[1]
    b_per_w = b // _NW
    blk = chunk * nbuf
    pair_blocks = 2 * (b_per_w // blk)
    if k0 is None:
        k0 = pair_blocks // 2

    @functools.partial(
        pl.kernel, mesh=_MESH,
        out_type=jax.ShapeDtypeStruct((b, d), table.dtype),
        scratch_types=[
            pltpu.VMEM((blk,), jnp.int32),
            pltpu.VMEM((nbuf, chunk, d), table.dtype),
            pltpu.SemaphoreType.DMA,
            pltpu.SemaphoreType.DMA,
        ],
    )
    def k(table_hbm, idx_hbm, out_hbm, idx_v, rows_v, semg, semo):
        c = lax.axis_index("c")
        s = lax.axis_index("s")
        nblk = jnp.where(c == 0, k0, pair_blocks - k0)
        base = pl.multiple_of(
            s * 2 * b_per_w + jnp.where(c == 0, 0, k0 * blk), chunk)

        def body(i, carry):
            off = base + i * blk
            pltpu.sync_copy(idx_hbm.at[pl.ds(off, blk)], idx_v)
            gd = [pltpu.async_copy(
                table_hbm.at[idx_v.at[pl.ds(j * chunk, chunk)]],
                rows_v.at[j], semg) for j in range(nbuf)]
            od = []
            for j in range(nbuf):
                gd[j].wait()
                od.append(pltpu.async_copy(
                    rows_v.at[j],
                    out_hbm.at[pl.ds(off + j * chunk, chunk)], semo))
            for o in od:
                o.wait()
            return carry

        lax.fori_loop(0, nblk, body, 0)

    return k(table, idx)


def _sc_gather_sum5(table, idx, k0=None):
    b = idx.shape[0]
    d = table.shape[1]
    chunk = 80
    nbuf = 5
    blk = chunk * nbuf
    grp = chunk // _TZ
    b_per_w = b // _NW
    g_per_w = b_per_w // _TZ
    pair_blocks = 2 * (b_per_w // blk)
    if k0 is None:
        k0 = pair_blocks // 2

    @functools.partial(
        pl.kernel, mesh=_MESH,
        out_type=jax.ShapeDtypeStruct((b // _TZ, d), table.dtype),
        scratch_types=[
            pltpu.VMEM((blk,), jnp.int32),
            pltpu.VMEM((nbuf, chunk, d), table.dtype),
            pltpu.VMEM((blk // _TZ, d), table.dtype),
            pltpu.SemaphoreType.DMA,
        ],
    )
    def k(table_hbm, idx_hbm, out_hbm, idx_v, rows_v, sum_v, sem):
        c = lax.axis_index("c")
        s = lax.axis_index("s")
        nblk = jnp.where(c == 0, k0, pair_blocks - k0)
        base = pl.multiple_of(
            s * 2 * b_per_w + jnp.where(c == 0, 0, k0 * blk), 80)
        gbase = pl.multiple_of(base // _TZ, 16)

        def body(i, carry):
            pltpu.sync_copy(idx_hbm.at[pl.ds(base + i * blk, blk)], idx_v)
            gd = [pltpu.async_copy(
                table_hbm.at[idx_v.at[pl.ds(j * chunk, chunk)]],
                rows_v.at[j], sem) for j in range(nbuf)]
            for j in range(nbuf):
                gd[j].wait()

                def gsum(g, carry2):
                    for cb in range(d // 16):
                        sl = pl.ds(cb * 16, 16)
                        acc = rows_v[j, 5 * g, sl]
                        for z in range(1, _TZ):
                            acc = acc + rows_v[j, 5 * g + z, sl]
                        sum_v[j * grp + g, sl] = acc
                    return carry2

                lax.fori_loop(0, grp, gsum, 0)
            pltpu.sync_copy(sum_v,
                            out_hbm.at[pl.ds(gbase + i * (blk // _TZ),
                                             blk // _TZ)])
            return carry

        lax.fori_loop(0, nblk, body, 0)

    return k(table, idx)


_HN = _NP // 2
_ACC = 5248
_EC = 128
_ENB = 4
_ZC = _ACC // _NS // 4


def _zero_acc(zeros_hbm, z_v, acc_sh, s):
    pltpu.sync_copy(zeros_hbm, z_v)
    for q in range(4):
        pltpu.sync_copy(z_v, acc_sh.at[pl.ds((s * 4 + q) * _ZC, _ZC)])


def _remap_local(dst_v, j, lo):
    for kk in range(_EC // 16):
        sl = pl.ds(kk * 16, 16)
        loc = dst_v[j, sl] - lo
        ok = (loc >= 0) & (loc < _HN)
        dst_v[j, sl] = jnp.where(ok, loc, _HN)


def _sc_degree(dst2, ones_rows, zeros_rows):
    rows_pt = dst2.shape[0] // _NS
    nblk = rows_pt // _ENB
    orows_t = _HN // _NS

    @functools.partial(
        pl.kernel, mesh=_MESH,
        out_type=jax.ShapeDtypeStruct((_NC, _HN, _D), jnp.float32),
        scratch_types=[
            pltpu.VMEM((_ENB, _EC), jnp.int32),
            pltpu.VMEM((_EC, _D), jnp.float32),
            pltpu.VMEM((_ZC, _D), jnp.float32),
            pltpu.VMEM_SHARED((_ACC, _D), jnp.float32),
            pltpu.SemaphoreType.DMA,
        ],
    )
    def k(dst_hbm, ones_hbm, zeros_hbm, out_hbm, dst_v, ones_v, z_v, acc_sh,
          sem):
        c = lax.axis_index("c")
        s = lax.axis_index("s")
        base = s * rows_pt
        lo = c * _HN
        pltpu.sync_copy(ones_hbm, ones_v)
        _zero_acc(zeros_hbm, z_v, acc_sh, s)
        plsc.subcore_barrier()

        def body(i, carry):
            pltpu.sync_copy(dst_hbm.at[pl.ds(base + i * _ENB, _ENB)], dst_v)
            sd = []
            for j in range(_ENB):
                _remap_local(dst_v, j, lo)
                sd.append(pltpu.async_copy(
                    ones_v, acc_sh.at[dst_v.at[j]], sem, add=True))
            for o in sd:
                o.wait()
            return carry

        lax.fori_loop(0, nblk, body, 0)
        plsc.subcore_barrier()
        pltpu.sync_copy(acc_sh.at[pl.ds(s * orows_t, orows_t)],
                        out_hbm.at[c, pl.ds(s * orows_t, orows_t)])

    return k(dst2, ones_rows, zeros_rows)


def _sc_edge_segsum(g, src2, dst2, zeros_rows):
    d = g.shape[1]
    rows_pt = src2.shape[0] // _NS
    nblk = rows_pt // _ENB
    orows_t = _HN // _NS

    @functools.partial(
        pl.kernel, mesh=_MESH,
        out_type=jax.ShapeDtypeStruct((_NC, _HN, d), jnp.float32),
        scratch_types=[
            pltpu.VMEM((_ENB, _EC), jnp.int32),
            pltpu.VMEM((_ENB, _EC), jnp.int32),
            pltpu.VMEM((_ENB, _EC, d), jnp.float32),
            pltpu.VMEM((_ZC, d), jnp.float32),
            pltpu.VMEM_SHARED((_ACC, d), jnp.float32),
            pltpu.SemaphoreType.DMA,
            pltpu.SemaphoreType.DMA,
        ],
    )
    def k(g_hbm, src_hbm, dst_hbm, zeros_hbm, out_hbm,
          src_v, dst_v, rows_v, z_v, acc_sh, semg, semo):
        c = lax.axis_index("c")
        s = lax.axis_index("s")
        base = s * rows_pt
        lo = c * _HN
        _zero_acc(zeros_hbm, z_v, acc_sh, s)
        plsc.subcore_barrier()

        def body(i, carry):
            off = base + i * _ENB
            pltpu.sync_copy(src_hbm.at[pl.ds(off, _ENB)], src_v)
            pltpu.sync_copy(dst_hbm.at[pl.ds(off, _ENB)], dst_v)
            gd = [pltpu.async_copy(g_hbm.at[src_v.at[j]], rows_v.at[j], semg)
                  for j in range(_ENB)]
            sd = []
            for j in range(_ENB):
                _remap_local(dst_v, j, lo)
                gd[j].wait()
                sd.append(pltpu.async_copy(
                    rows_v.at[j], acc_sh.at[dst_v.at[j]], semo, add=True))
            for o in sd:
                o.wait()
            return carry

        lax.fori_loop(0, nblk, body, 0)
        plsc.subcore_barrier()
        pltpu.sync_copy(acc_sh.at[pl.ds(s * orows_t, orows_t)],
                        out_hbm.at[c, pl.ds(s * orows_t, orows_t)])

    return k(g, src2, dst2, zeros_rows)


_BLK = 128


def _tc_conv_kernel(e_ref, wcat_ref, b1_ref, b2_ref, x_ref):
    eb = e_ref[...].astype(jnp.bfloat16)
    a = jnp.dot(eb, wcat_ref[...], preferred_element_type=jnp.float32)
    a3 = a.reshape(_BLK, _SEQ, 6 * _F)
    p1 = a3[:, :_SEQ - 1, 0:_F] + a3[:, 1:, _F:2 * _F]
    m1 = jnp.max(p1, axis=1)
    p2 = (a3[:, :_SEQ - 3, 2 * _F:3 * _F] + a3[:, 1:_SEQ - 2, 3 * _F:4 * _F]
          + a3[:, 2:_SEQ - 1, 4 * _F:5 * _F] + a3[:, 3:, 5 * _F:6 * _F])
    m2 = jnp.max(p2, axis=1)
    x1 = jnp.maximum(m1 + b1_ref[...], 0.0)
    x2 = jnp.maximum(m2 + b2_ref[...], 0.0)
    x_ref[...] = jnp.concatenate([x1, x2], axis=1)


def _tc_conv(e2, wcat, b1, b2):
    n_blk = _NP // _BLK
    return pl.pallas_call(
        _tc_conv_kernel,
        grid=(n_blk,),
        in_specs=[
            pl.BlockSpec((_BLK * _SEQ, _D), lambda i: (i, 0)),
            pl.BlockSpec((_D, 6 * _F), lambda i: (0, 0)),
            pl.BlockSpec((1, _F), lambda i: (0, 0)),
            pl.BlockSpec((1, _F), lambda i: (0, 0)),
        ],
        out_specs=pl.BlockSpec((_BLK, 2 * _F), lambda i: (i, 0)),
        out_shape=jax.ShapeDtypeStruct((_NP, 2 * _F), jnp.float32),
    )(e2, wcat, b1, b2)


def _tc_tagfc_kernel(ts_ref, fc1t_ref, fc2t_ref, t_ref):
    ts = ts_ref[...]
    r1 = jnp.maximum(jnp.dot(ts, fc1t_ref[...],
                             preferred_element_type=jnp.float32), 0.0)
    r2 = jnp.maximum(jnp.dot(ts, fc2t_ref[...],
                             preferred_element_type=jnp.float32), 0.0)
    r1 = r1.reshape(_BLK, _NTAG, _F)
    r2 = r2.reshape(_BLK, _NTAG, _F)
    t1 = jnp.sum(r1, axis=1)
    t2 = jnp.sum(r2, axis=1)
    t_ref[...] = jnp.concatenate([t1, t2], axis=1)


def _tc_tagfc(ts2, fc1t, fc2t):
    n_blk = _NP // _BLK
    return pl.pallas_call(
        _tc_tagfc_kernel,
        grid=(n_blk,),
        in_specs=[
            pl.BlockSpec((_BLK * _NTAG, _D), lambda i: (i, 0)),
            pl.BlockSpec((_D, _F), lambda i: (0, 0)),
            pl.BlockSpec((_D, _F), lambda i: (0, 0)),
        ],
        out_specs=pl.BlockSpec((_BLK, 2 * _F), lambda i: (i, 0)),
        out_shape=jax.ShapeDtypeStruct((_NP, 2 * _F), jnp.float32),
    )(ts2, fc1t, fc2t)


def _bn_cols(x, mask, gamma, beta):
    cnt = float(_N)
    xm = jnp.sum(x * mask, axis=0, keepdims=True) / cnt
    xv = jnp.sum(x * x * mask, axis=0, keepdims=True) / cnt - xm * xm
    return (x - xm) * lax.rsqrt(xv + 1e-5) * gamma + beta


def _dinv_col(degp_ref):
    deg = jnp.concatenate(
        [degp_ref[0, :, 0:1], degp_ref[1, :, 0:1]], axis=0) + 1.0
    return lax.rsqrt(deg)


def _tc_bn_combine_kernel(x_ref, t_ref, degp_ref, g1_ref, be1_ref,
                          g2_ref, be2_ref, w1_ref, hw_ref, gv_ref):
    mask = (lax.broadcasted_iota(jnp.int32, (_NP, 1), 0) < _N
            ).astype(jnp.float32)
    xb = _bn_cols(x_ref[...], mask, g1_ref[...], be1_ref[...])
    tb = _bn_cols(t_ref[...], mask, g2_ref[...], be2_ref[...])
    h = xb + tb
    dinv = _dinv_col(degp_ref)
    hw = jnp.dot(h, w1_ref[...], preferred_element_type=jnp.float32)
    hw_ref[...] = hw
    gv_ref[...] = hw * dinv


def _tc_bn_combine(x, t, degp, bn1g, bn1b, bn2g, bn2b, gcn1_w):
    sds = jax.ShapeDtypeStruct((_NP, _D), jnp.float32)
    return pl.pallas_call(
        _tc_bn_combine_kernel,
        out_shape=[sds, sds],
    )(x, t, degp, bn1g, bn1b, bn2g, bn2b, gcn1_w)


def _tc_gcn_mid_kernel(s_ref, hw_ref, degp_ref, b_ref, w2_ref,
                       hw2_ref, gv_ref):
    dinv = _dinv_col(degp_ref)
    agg = jnp.concatenate([s_ref[0], s_ref[1]], axis=0)
    h1 = dinv * agg + dinv * dinv * hw_ref[...] + b_ref[...]
    h1 = jnp.maximum(h1, 0.0)
    hw2 = jnp.dot(h1, w2_ref[...], preferred_element_type=jnp.float32)
    hw2_ref[...] = hw2
    gv_ref[...] = hw2 * dinv


def _tc_gcn_mid(s, hw, degp, b, w2):
    sds = jax.ShapeDtypeStruct((_NP, _D), jnp.float32)
    return pl.pallas_call(
        _tc_gcn_mid_kernel,
        out_shape=[sds, sds],
    )(s, hw, degp, b, w2)


def _tc_gcn_out_kernel(s_ref, hw_ref, degp_ref, b_ref, emb_ref):
    dinv = _dinv_col(degp_ref)
    agg = jnp.concatenate([s_ref[0], s_ref[1]], axis=0)
    h2 = dinv * agg + dinv * dinv * hw_ref[...] + b_ref[...]
    nrm = jnp.sqrt(jnp.sum(h2 * h2, axis=1, keepdims=True))
    emb_ref[...] = h2 / jnp.maximum(nrm, 1e-12)


def _tc_gcn_out(s, hw, degp, b):
    return pl.pallas_call(
        _tc_gcn_out_kernel,
        out_shape=jax.ShapeDtypeStruct((_NP, _D), jnp.float32),
    )(s, hw, degp, b)


def kernel(repo_index, repo_features, repo_tags, bridge_ids, repo_edge_index,
           embed_table, conv1_w, conv1_b, conv2_w, conv2_b, bn1_gamma,
           bn1_beta, fc1_w, fc2_w, bn2_gamma, bn2_beta, gcn1_w, gcn1_b,
           gcn2_w, gcn2_b):
    bn, kk = repo_index.shape

    feat_idx = jnp.concatenate(
        [repo_features,
         jnp.zeros((_NP - _N, _SEQ), jnp.int32)], axis=0).reshape(-1)
    tag_idx = jnp.concatenate(
        [repo_tags,
         jnp.zeros((_NP - _N, _NTAG, _TZ), jnp.int32)], axis=0).reshape(-1)
    src2 = jnp.concatenate(
        [repo_edge_index[0],
         jnp.zeros((_NEP - _NE,), jnp.int32)]).reshape(_NEP // _EC, _EC)
    dst2 = jnp.concatenate(
        [repo_edge_index[1],
         jnp.full((_NEP - _NE,), _N, jnp.int32)]).reshape(_NEP // _EC, _EC)

    wcat = jnp.concatenate(
        [conv1_w[:, 0, 0, :].T, conv1_w[:, 0, 1, :].T,
         conv2_w[:, 0, 0, :].T, conv2_w[:, 0, 1, :].T,
         conv2_w[:, 0, 2, :].T, conv2_w[:, 0, 3, :].T], axis=1)
    b1 = conv1_b.reshape(1, _F)
    b2 = conv2_b.reshape(1, _F)
    fc1t = fc1_w.T
    fc2t = fc2_w.T
    bn1g = bn1_gamma.reshape(1, 2 * _F)
    bn1b = bn1_beta.reshape(1, 2 * _F)
    bn2g = bn2_gamma.reshape(1, 2 * _F)
    bn2b = bn2_beta.reshape(1, 2 * _F)
    g1b = gcn1_b.reshape(1, _D)
    g2b = gcn2_b.reshape(1, _D)

    zeros_acc = jnp.zeros((_ZC, _D), jnp.float32)
    ones_d = jnp.ones((_EC, _D), jnp.float32)

    e2 = _sc_gather(embed_table, feat_idx, 128, 5, k0=40)
    x = _tc_conv(e2, wcat.astype(jnp.bfloat16), b1, b2)
    dep = (e2[0, 0] * 0.0).astype(jnp.int32)
    ts2 = _sc_gather_sum5(embed_table, tag_idx + dep, k0=59)
    t = _tc_tagfc(ts2, fc1t, fc2t)
    degp = _sc_degree(dst2, ones_d, zeros_acc)

    hw1, g1v = _tc_bn_combine(x, t, degp, bn1g, bn1b, bn2g, bn2b, gcn1_w)

    s1 = _sc_edge_segsum(g1v, src2, dst2, zeros_acc)
    hw2, g2v = _tc_gcn_mid(s1, hw1, degp, g1b, gcn2_w)

    s2 = _sc_edge_segsum(g2v, src2, dst2, zeros_acc)
    all_emb = _tc_gcn_out(s2, hw2, degp, g2b)

    fin_idx = jnp.concatenate([bridge_ids, repo_index.reshape(-1)])
    fin = _sc_gather(all_emb, fin_idx, 96, 6)
    bridges = fin[:bridge_ids.shape[0]]
    sel = fin[bridge_ids.shape[0]:].reshape(bn, kk, _D)
    return bridges, sel
